# baseline (speedup 1.0000x reference)
import functools

import numpy as np
import jax
import jax.numpy as jnp
from jax import lax
from jax.experimental import pallas as pl
from jax.experimental.pallas import tpu as pltpu
from jax.experimental.pallas import tpu_sc as plsc

N = 320000
D = 128
S = 1024

NC = 2
NS = 16
NW = NC * NS

CHUNK = 128
NCHUNKS = N // CHUNK
BASE_PER = NCHUNKS // NW
EXTRA = NCHUNKS - BASE_PER * NW
BLK_CHUNKS = 3
BLK = BLK_CHUNKS * CHUNK
NBLK = BASE_PER // BLK_CHUNKS
OUT_ROWS = S // NS

_ROW_IDS = np.zeros((NW, BASE_PER + 1), dtype=np.int32)
for _w in range(NW):
    _ROW_IDS[_w, :BASE_PER] = _w * BASE_PER + np.arange(BASE_PER)
    _ROW_IDS[_w, BASE_PER] = NW * BASE_PER + min(_w, EXTRA - 1)

_mesh = plsc.VectorSubcoreMesh(core_axis_name="c", subcore_axis_name="s")


@functools.partial(
    pl.kernel,
    mesh=_mesh,
    out_type=jax.ShapeDtypeStruct((NC, S, D), jnp.float32),
    scratch_types=[
        pltpu.VMEM((2, BLK, D), jnp.float32),
        pltpu.VMEM((BASE_PER + 1, CHUNK), jnp.int32),
        pltpu.VMEM((OUT_ROWS, D), jnp.float32),
        pltpu.VMEM_SHARED((S, D), jnp.float32),
        pltpu.SemaphoreType.DMA,
        pltpu.SemaphoreType.DMA,
        pltpu.SemaphoreType.DMA,
        pltpu.SemaphoreType.DMA,
    ],
)
def _segsum_sc(v_hbm, batch_hbm, zeros_hbm, out_hbm,
               vbuf, ibuf, obuf, acc, sem0, sem1, sems0, sems1):
    c = lax.axis_index("c")
    s = lax.axis_index("s")
    wid = s * NC + c
    row0 = wid * (BASE_PER * CHUNK)

    def vload(g, slot, sem):
        return pltpu.make_async_copy(
            v_hbm.at[pl.ds(row0 + g * BLK, BLK), :], vbuf.at[slot], sem)

    vload(0, 0, sem0).start()
    vload(1, 1, sem1).start()
    pltpu.sync_copy(batch_hbm.at[wid], ibuf)
    pltpu.sync_copy(zeros_hbm, obuf)
    pltpu.sync_copy(obuf, acc.at[pl.ds(s * OUT_ROWS, OUT_ROWS)])
    plsc.subcore_barrier()

    def scatter_block(g, slot, sem):
        return [
            pltpu.async_copy(vbuf.at[slot, pl.ds(k * CHUNK, CHUNK)],
                             acc.at[ibuf.at[g * BLK_CHUNKS + k]], sem,
                             add=True)
            for k in range(BLK_CHUNKS)
        ]

    def body(gg, carry):
        g0 = 2 * gg
        g1 = g0 + 1
        vload(g0, 0, sem0).wait()
        h0 = scatter_block(g0, 0, sems0)
        vload(g1, 1, sem1).wait()
        h1 = scatter_block(g1, 1, sems1)

        for h in h0:
            h.wait()

        @pl.when(g0 + 2 < NBLK)
        def _():
            vload(g0 + 2, 0, sem0).start()

        for h in h1:
            h.wait()

        @pl.when(g1 + 2 < NBLK)
        def _():
            vload(g1 + 2, 1, sem1).start()

        return carry

    lax.fori_loop(0, NBLK // 2, body, 0)

    @pl.when(wid < EXTRA)
    def _():
        off = (NW * BASE_PER + wid) * CHUNK
        pltpu.sync_copy(v_hbm.at[pl.ds(off, CHUNK), :],
                        vbuf.at[0, pl.ds(0, CHUNK)])
        pltpu.sync_copy(vbuf.at[0, pl.ds(0, CHUNK)],
                        acc.at[ibuf.at[BASE_PER]], add=True)

    plsc.subcore_barrier()

    pltpu.sync_copy(acc.at[pl.ds(s * OUT_ROWS, OUT_ROWS)], obuf)
    pltpu.sync_copy(obuf, out_hbm.at[c, pl.ds(s * OUT_ROWS, OUT_ROWS), :])


def _combine_body(u_ref, p_ref, o_ref):
    o_ref[...] = u_ref[...] + p_ref[0] + p_ref[1]


def kernel(u, v, batch):
    batch2d = batch.astype(jnp.int32).reshape(NCHUNKS, CHUNK)
    batch3d = batch2d[jnp.asarray(_ROW_IDS)]
    zeros = jnp.zeros((OUT_ROWS, D), jnp.float32)
    partials = _segsum_sc(v, batch3d, zeros)
    return pl.pallas_call(
        _combine_body,
        out_shape=jax.ShapeDtypeStruct((S, D), jnp.float32),
    )(u, partials)


# speedup vs baseline: 1.2870x; 1.2870x over previous
import functools

import jax
import jax.numpy as jnp
from jax import lax
from jax.experimental import pallas as pl
from jax.experimental.pallas import tpu as pltpu
from jax.experimental.pallas import tpu_sc as plsc

N = 320000
D = 128
S = 1024

NC = 2
NS = 16
NW = NC * NS

CHUNK = 128
NCHUNKS = N // CHUNK
BASE_PER = NCHUNKS // NW
EXTRA = NCHUNKS - BASE_PER * NW
BLK_CHUNKS = 3
BLK = BLK_CHUNKS * CHUNK
NBLK = BASE_PER // BLK_CHUNKS
OUT_ROWS = S // NS

IB_MAIN = 88
LAST_START = 2408
EXTRA_START = NW * BASE_PER

_mesh = plsc.VectorSubcoreMesh(core_axis_name="c", subcore_axis_name="s")


@functools.partial(
    pl.kernel,
    mesh=_mesh,
    out_type=jax.ShapeDtypeStruct((NC, S, D), jnp.float32),
    scratch_types=[
        pltpu.VMEM((2, BLK, D), jnp.float32),
        pltpu.VMEM((IB_MAIN + 4, CHUNK), jnp.int32),
        pltpu.VMEM((OUT_ROWS, D), jnp.float32),
        pltpu.VMEM_SHARED((S, D), jnp.float32),
        pltpu.SemaphoreType.DMA,
        pltpu.SemaphoreType.DMA,
    ],
)
def _segsum_sc(v_hbm, batch_hbm, zeros_hbm, out_hbm,
               vbuf, ibuf, obuf, acc, sem0, sem1):
    c = lax.axis_index("c")
    s = lax.axis_index("s")
    wid = s * NC + c
    row0 = wid * (BASE_PER * CHUNK)

    def vload(g, slot, sem):
        return pltpu.make_async_copy(
            v_hbm.at[pl.ds(row0 + g * BLK, BLK), :], vbuf.at[slot], sem)

    vload(0, 0, sem0).start()
    vload(1, 1, sem1).start()
    chunk0 = wid * BASE_PER
    ib_start = pl.multiple_of(
        jnp.minimum((chunk0 // 8) * 8, LAST_START), 8)
    ib_off = chunk0 - ib_start
    pltpu.sync_copy(batch_hbm.at[pl.ds(ib_start, IB_MAIN)],
                    ibuf.at[pl.ds(0, IB_MAIN)])
    pltpu.sync_copy(batch_hbm.at[pl.ds(EXTRA_START, EXTRA)],
                    ibuf.at[pl.ds(IB_MAIN, EXTRA)])
    pltpu.sync_copy(zeros_hbm, obuf)
    pltpu.sync_copy(obuf, acc.at[pl.ds(s * OUT_ROWS, OUT_ROWS)])
    plsc.subcore_barrier()

    def scatter_block(g, slot):
        for k in range(BLK_CHUNKS):
            pltpu.sync_copy(vbuf.at[slot, pl.ds(k * CHUNK, CHUNK)],
                            acc.at[ibuf.at[ib_off + g * BLK_CHUNKS + k]],
                            add=True)

    def body(gg, carry):
        g0 = 2 * gg
        vload(g0, 0, sem0).wait()
        scatter_block(g0, 0)

        @pl.when(g0 + 2 < NBLK)
        def _():
            vload(g0 + 2, 0, sem0).start()

        vload(g0 + 1, 1, sem1).wait()
        scatter_block(g0 + 1, 1)

        @pl.when(g0 + 3 < NBLK)
        def _():
            vload(g0 + 3, 1, sem1).start()

        return carry

    lax.fori_loop(0, NBLK // 2, body, 0)

    @pl.when(wid < EXTRA)
    def _():
        off = (EXTRA_START + wid) * CHUNK
        pltpu.sync_copy(v_hbm.at[pl.ds(off, CHUNK), :],
                        vbuf.at[0, pl.ds(0, CHUNK)])
        pltpu.sync_copy(vbuf.at[0, pl.ds(0, CHUNK)],
                        acc.at[ibuf.at[IB_MAIN + wid]], add=True)

    plsc.subcore_barrier()

    pltpu.sync_copy(acc.at[pl.ds(s * OUT_ROWS, OUT_ROWS)], obuf)
    pltpu.sync_copy(obuf, out_hbm.at[c, pl.ds(s * OUT_ROWS, OUT_ROWS), :])


def _combine_body(u_ref, p_ref, o_ref):
    o_ref[...] = u_ref[...] + p_ref[0] + p_ref[1]


def kernel(u, v, batch):
    batch2d = batch.astype(jnp.int32).reshape(NCHUNKS, CHUNK)
    zeros = jnp.zeros((OUT_ROWS, D), jnp.float32)
    partials = _segsum_sc(v, batch2d, zeros)
    return pl.pallas_call(
        _combine_body,
        out_shape=jax.ShapeDtypeStruct((S, D), jnp.float32),
    )(u, partials)


# speedup vs baseline: 1.5281x; 1.1873x over previous
import functools

import jax
import jax.numpy as jnp
from jax import lax
from jax.experimental import pallas as pl
from jax.experimental.pallas import tpu as pltpu
from jax.experimental.pallas import tpu_sc as plsc

N = 320000
D = 128
S = 1024

NC = 2
NS = 16
NW = NC * NS

CHUNK = 128
NCHUNKS = N // CHUNK
BASE_PER = NCHUNKS // NW
EXTRA = NCHUNKS - BASE_PER * NW
BLK_CHUNKS = 3
BLK = BLK_CHUNKS * CHUNK
NBLK = BASE_PER // BLK_CHUNKS
OUT_ROWS = S // NS
LANES = 16
NSEG_V = D // LANES
TRASH = S

IB_MAIN = 88
LAST_START = 2408
EXTRA_START = NW * BASE_PER

_mesh = plsc.VectorSubcoreMesh(core_axis_name="c", subcore_axis_name="s")


@functools.partial(
    pl.kernel,
    mesh=_mesh,
    out_type=jax.ShapeDtypeStruct((NC, S, D), jnp.float32),
    scratch_types=[
        pltpu.VMEM((2, BLK, D), jnp.float32),
        pltpu.VMEM((IB_MAIN + 4, CHUNK), jnp.int32),
        pltpu.VMEM((OUT_ROWS, D), jnp.float32),
        pltpu.VMEM((LANES, D), jnp.float32),
        pltpu.VMEM((LANES,), jnp.int32),
        pltpu.VMEM_SHARED((S + 8, D), jnp.float32),
        pltpu.SemaphoreType.DMA,
        pltpu.SemaphoreType.DMA,
    ],
)
def _segsum_sc(v_hbm, batch_hbm, zeros_hbm, out_hbm,
               vbuf, ibuf, obuf, sbuf, idbuf, acc, sem0, sem1):
    c = lax.axis_index("c")
    s = lax.axis_index("s")
    wid = s * NC + c
    row0 = wid * (BASE_PER * CHUNK)
    lane_iota = lax.iota(jnp.int32, LANES)

    def vload(g, slot, sem):
        return pltpu.make_async_copy(
            v_hbm.at[pl.ds(row0 + g * BLK, BLK), :], vbuf.at[slot], sem)

    vload(0, 0, sem0).start()
    vload(1, 1, sem1).start()
    chunk0 = wid * BASE_PER
    ib_start = pl.multiple_of(
        jnp.minimum((chunk0 // 8) * 8, LAST_START), 8)
    ib_off = chunk0 - ib_start
    pltpu.sync_copy(batch_hbm.at[pl.ds(ib_start, IB_MAIN)],
                    ibuf.at[pl.ds(0, IB_MAIN)])
    pltpu.sync_copy(batch_hbm.at[pl.ds(EXTRA_START, EXTRA)],
                    ibuf.at[pl.ds(IB_MAIN, EXTRA)])
    pltpu.sync_copy(zeros_hbm, obuf)
    pltpu.sync_copy(obuf, acc.at[pl.ds(s * OUT_ROWS, OUT_ROWS)])
    plsc.subcore_barrier()

    def reduce_chunk(slot, base):
        def rbody(r, accs):
            out = accs
            for rr in range(4):
                row = base + 4 * r + rr
                out = tuple(
                    a + jnp.reshape(
                        vbuf[slot, pl.ds(row, 1), pl.ds(j * LANES, LANES)],
                        (LANES,))
                    for j, a in enumerate(out))
            return out

        zero = tuple(jnp.zeros((LANES,), jnp.float32) for _ in range(NSEG_V))
        return lax.fori_loop(0, CHUNK // 4, rbody, zero)

    def fold_block(g, slot):
        idbuf[...] = jnp.full((LANES,), TRASH, jnp.int32)
        for k in range(BLK_CHUNKS):
            irow = ib_off + g * BLK_CHUNKS + k
            fv = jnp.reshape(ibuf[pl.ds(irow, 1), pl.ds(0, LANES)],
                             (LANES,))
            lv = jnp.reshape(
                ibuf[pl.ds(irow, 1), pl.ds(CHUNK - LANES, LANES)],
                (LANES,))
            single = fv[0] == lv[LANES - 1]

            @pl.when(single)
            def _():
                sums = reduce_chunk(slot, k * CHUNK)
                for j in range(NSEG_V):
                    sbuf[k, pl.ds(j * LANES, LANES)] = sums[j]
                idbuf[...] = jnp.where(lane_iota == k, fv, idbuf[...])

            @pl.when(jnp.logical_not(single))
            def _():
                pltpu.sync_copy(vbuf.at[slot, pl.ds(k * CHUNK, CHUNK)],
                                acc.at[ibuf.at[irow]], add=True)

        pltpu.sync_copy(sbuf, acc.at[idbuf], add=True)

    def body(gg, carry):
        g0 = 2 * gg
        vload(g0, 0, sem0).wait()
        fold_block(g0, 0)

        @pl.when(g0 + 2 < NBLK)
        def _():
            vload(g0 + 2, 0, sem0).start()

        vload(g0 + 1, 1, sem1).wait()
        fold_block(g0 + 1, 1)

        @pl.when(g0 + 3 < NBLK)
        def _():
            vload(g0 + 3, 1, sem1).start()

        return carry

    lax.fori_loop(0, NBLK // 2, body, 0)

    @pl.when(wid < EXTRA)
    def _():
        off = (EXTRA_START + wid) * CHUNK
        pltpu.sync_copy(v_hbm.at[pl.ds(off, CHUNK), :],
                        vbuf.at[0, pl.ds(0, CHUNK)])
        pltpu.sync_copy(vbuf.at[0, pl.ds(0, CHUNK)],
                        acc.at[ibuf.at[IB_MAIN + wid]], add=True)

    plsc.subcore_barrier()

    pltpu.sync_copy(acc.at[pl.ds(s * OUT_ROWS, OUT_ROWS)], obuf)
    pltpu.sync_copy(obuf, out_hbm.at[c, pl.ds(s * OUT_ROWS, OUT_ROWS), :])


def _combine_body(u_ref, p_ref, o_ref):
    o_ref[...] = u_ref[...] + p_ref[0] + p_ref[1]


def kernel(u, v, batch):
    batch2d = batch.astype(jnp.int32).reshape(NCHUNKS, CHUNK)
    zeros = jnp.zeros((OUT_ROWS, D), jnp.float32)
    partials = _segsum_sc(v, batch2d, zeros)
    return pl.pallas_call(
        _combine_body,
        out_shape=jax.ShapeDtypeStruct((S, D), jnp.float32),
    )(u, partials)
